# baseline (speedup 1.0000x reference)
import functools

import jax
import jax.numpy as jnp
from jax import lax
from jax.experimental import pallas as pl
from jax.experimental.pallas import tpu as pltpu
from jax.experimental.pallas import tpu_sc as plsc

E = 8
K = 2
N = 2048
D = 768
F = 2048
T = 256
P = N * K
S = ((P + E * (T - 1) + T - 1) // T) * T
NB = S // T

NC, NS = 2, 16
NW = NC * NS


def _router_body(x_ref, wr_ref, w_ref, i_ref):
    logits = jnp.dot(x_ref[...], wr_ref[...], preferred_element_type=jnp.float32)
    m = jnp.max(logits, axis=-1, keepdims=True)
    p = jnp.exp(logits - m)
    p = p / jnp.sum(p, axis=-1, keepdims=True)
    iota = lax.broadcasted_iota(jnp.int32, p.shape, 1)
    m1 = jnp.max(p, axis=-1, keepdims=True)
    i1 = jnp.min(jnp.where(p == m1, iota, E), axis=-1, keepdims=True)
    p2 = jnp.where(iota == i1, -1.0, p)
    m2 = jnp.max(p2, axis=-1, keepdims=True)
    i2 = jnp.min(jnp.where(p2 == m2, iota, E), axis=-1, keepdims=True)
    s = m1 + m2
    w_ref[...] = jnp.concatenate([m1 / s, m2 / s], axis=1)
    i_ref[...] = jnp.concatenate([i1, i2], axis=1)


def _router(x_flat, Wr):
    return pl.pallas_call(
        _router_body,
        out_shape=(
            jax.ShapeDtypeStruct((N, K), jnp.float32),
            jax.ShapeDtypeStruct((N, K), jnp.int32),
        ),
    )(x_flat, Wr)


GCH = 48


@functools.cache
def _sc_gather_kernel():
    mesh = plsc.VectorSubcoreMesh(
        core_axis_name="c", subcore_axis_name="s", num_cores=NC, num_subcores=NS
    )
    rows_per_w = S // NW

    @functools.partial(
        pl.kernel,
        mesh=mesh,
        out_type=jax.ShapeDtypeStruct((S, D), jnp.float32),
        scratch_types=[
            pltpu.VMEM((GCH,), jnp.int32),
            pltpu.VMEM((GCH, D), jnp.float32),
            pltpu.SemaphoreType.DMA,
        ],
    )
    def k(x_hbm, tok_hbm, out_hbm, idx_v, rows_v, sem):
        wid = lax.axis_index("s") * NC + lax.axis_index("c")
        base = pl.multiple_of(wid * rows_per_w, GCH)

        def body(c, carry):
            start = pl.multiple_of(base + c * GCH, 8)
            pltpu.sync_copy(tok_hbm.at[pl.ds(start, GCH)], idx_v)
            pltpu.async_copy(x_hbm.at[idx_v], rows_v, sem).wait()
            pltpu.sync_copy(rows_v, out_hbm.at[pl.ds(start, GCH)])
            return carry

        lax.fori_loop(0, rows_per_w // GCH, body, 0)

    return k


def _mlp_body(be_ref, xs_ref, w1_ref, b1_ref, w2_ref, b2_ref, ws_ref, ys_ref):
    h = jax.nn.gelu(
        jnp.dot(xs_ref[...], w1_ref[0], preferred_element_type=jnp.float32)
        + b1_ref[0]
    )
    y = jnp.dot(h, w2_ref[0], preferred_element_type=jnp.float32)
    ys_ref[...] = (y + b2_ref[0]) * ws_ref[...]


def _grouped_mlp(block_expert, Xs, W1, b1, W2, b2, w_slot):
    grid_spec = pltpu.PrefetchScalarGridSpec(
        num_scalar_prefetch=1,
        grid=(NB,),
        in_specs=[
            pl.BlockSpec((T, D), lambda i, be: (i, 0)),
            pl.BlockSpec((1, D, F), lambda i, be: (be[i], 0, 0)),
            pl.BlockSpec((1, 1, F), lambda i, be: (be[i], 0, 0)),
            pl.BlockSpec((1, F, D), lambda i, be: (be[i], 0, 0)),
            pl.BlockSpec((1, 1, D), lambda i, be: (be[i], 0, 0)),
            pl.BlockSpec((T, 1), lambda i, be: (i, 0)),
        ],
        out_specs=pl.BlockSpec((T, D), lambda i, be: (i, 0)),
    )
    return pl.pallas_call(
        _mlp_body,
        grid_spec=grid_spec,
        out_shape=jax.ShapeDtypeStruct((S, D), jnp.float32),
    )(block_expert, Xs, W1, b1.reshape(E, 1, F), W2, b2.reshape(E, 1, D), w_slot)


CCH = 32


@functools.cache
def _sc_combine_kernel():
    mesh = plsc.VectorSubcoreMesh(
        core_axis_name="c", subcore_axis_name="s", num_cores=NC, num_subcores=NS
    )
    tok_per_w = N // NW

    @functools.partial(
        pl.kernel,
        mesh=mesh,
        out_type=jax.ShapeDtypeStruct((N, D), jnp.float32),
        scratch_types=[
            pltpu.VMEM((CCH,), jnp.int32),
            pltpu.VMEM((CCH,), jnp.int32),
            pltpu.VMEM((CCH, D), jnp.float32),
            pltpu.VMEM((CCH, D), jnp.float32),
            pltpu.SemaphoreType.DMA,
        ],
    )
    def k(ys_hbm, p0_hbm, p1_hbm, out_hbm, i0_v, i1_v, r0_v, r1_v, sem):
        wid = lax.axis_index("s") * NC + lax.axis_index("c")
        base = pl.multiple_of(wid * tok_per_w, CCH)

        def chunk(c, carry):
            start = pl.multiple_of(base + c * CCH, 8)
            pltpu.sync_copy(p0_hbm.at[pl.ds(start, CCH)], i0_v)
            pltpu.sync_copy(p1_hbm.at[pl.ds(start, CCH)], i1_v)
            cp0 = pltpu.async_copy(ys_hbm.at[i0_v], r0_v, sem)
            cp1 = pltpu.async_copy(ys_hbm.at[i1_v], r1_v, sem)
            cp0.wait()
            cp1.wait()

            def row(i, rcarry):
                for ch in range(D // 16):
                    sl = pl.ds(ch * 16, 16)
                    r0_v[i, sl] = r0_v[i, sl] + r1_v[i, sl]
                return rcarry

            lax.fori_loop(0, CCH, row, 0)
            pltpu.sync_copy(r0_v, out_hbm.at[pl.ds(start, CCH)])
            return carry

        lax.fori_loop(0, tok_per_w // CCH, chunk, 0)

    return k


def kernel(x, Wr, W1, b1, W2, b2):
    Bb, Ll, Dd = x.shape
    x_flat = x.reshape(Bb * Ll, Dd)

    w, idx = _router(x_flat, Wr)

    e = idx.reshape(P)
    wf = w.reshape(P)
    oh = (e[:, None] == jnp.arange(E, dtype=jnp.int32)[None, :]).astype(jnp.int32)
    csum = jnp.cumsum(oh, axis=0)
    rank = jnp.take_along_axis(csum, e[:, None], axis=1)[:, 0] - 1
    cnt = csum[-1]
    cnt_pad = ((cnt + T - 1) // T) * T
    pad_cum = jnp.cumsum(cnt_pad)
    pad_off = pad_cum - cnt_pad
    dest = (pad_off[e] + rank).astype(jnp.int32)
    row_token = (
        jnp.zeros((S,), jnp.int32)
        .at[dest].set(jnp.arange(P, dtype=jnp.int32) // K)
    )
    w_slot = jnp.zeros((S, 1), jnp.float32).at[dest, 0].set(wf)
    block_expert = jnp.minimum(
        jnp.searchsorted(pad_cum, jnp.arange(NB, dtype=jnp.int32) * T, side="right"),
        E - 1,
    ).astype(jnp.int32)
    pos = dest.reshape(N, K)
    pos0 = pos[:, 0]
    pos1 = pos[:, 1]

    Xs = _sc_gather_kernel()(x_flat, row_token)
    Ys = _grouped_mlp(block_expert, Xs, W1, b1, W2, b2, w_slot)
    out = _sc_combine_kernel()(Ys, pos0, pos1)
    return out.reshape(Bb, Ll, Dd)
